# Initial kernel scaffold; baseline (speedup 1.0000x reference)
import jax
import jax.numpy as jnp
from jax.experimental import pallas as pl


def kernel(hidden_states, gate_w, w1, b1, w2, b2, sw1, sb1, sw2, sb2):
    raise NotImplementedError("write your pallas kernel here")


# speedup vs baseline: 3.4160x; 3.4160x over previous
import functools
import math

import jax
import jax.numpy as jnp
from jax import lax
from jax.experimental import pallas as pl
from jax.experimental.pallas import tpu as pltpu
from jax.experimental.pallas import tpu_sc as plsc

S, D, DFF, E = 2048, 768, 3072, 8
T = 128
NB = 23
M_PAD = NB * T

_SC_CORES = 2
_SC_SUBCORES = 16
_NW = _SC_CORES * _SC_SUBCORES
_ROWS_PER_W = S // _NW


def _shift_down_cumsum(c, n):
  k = 1
  while k < n:
    c = c + jnp.concatenate([jnp.zeros((k,) + c.shape[1:], c.dtype), c[:-k]], axis=0)
    k *= 2
  return c


def _gate_body(x_ref, gw_ref, dst_ref, wt_ref, cnt_ref):
  x = x_ref[...]
  logits = lax.dot_general(x, gw_ref[...], (((1,), (1,)), ((), ())),
                           preferred_element_type=jnp.float32)
  lmax = jnp.max(logits, axis=1, keepdims=True)
  ssum = jnp.sum(jnp.exp(logits - lmax), axis=1, keepdims=True)
  wt_ref[...] = 1.0 / ssum

  iota_e = lax.broadcasted_iota(jnp.int32, (S, E), 1)
  is_max = logits == lmax
  eidx = jnp.min(jnp.where(is_max, iota_e, E), axis=1, keepdims=True)
  onehot = (iota_e == eidx).astype(jnp.float32)

  cum = _shift_down_cumsum(onehot, S)
  counts = cum[S - 1:S, :]
  rank = jnp.sum(cum * onehot, axis=1, keepdims=True) - 1.0

  pc = jnp.floor((counts + (T - 1)) / T) * T
  inc = pc
  k = 1
  while k < E:
    inc = inc + jnp.concatenate([jnp.zeros((1, k), inc.dtype), inc[:, :-k]], axis=1)
    k *= 2
  starts = inc - pc

  dst = jnp.sum(onehot * starts, axis=1, keepdims=True) + rank
  dst_ref[...] = dst.astype(jnp.int32)
  cnt_ref[...] = counts.astype(jnp.int32)


def _gate(x, gate_w):
  return pl.pallas_call(
      _gate_body,
      out_shape=(
          jax.ShapeDtypeStruct((S, 1), jnp.int32),
          jax.ShapeDtypeStruct((S, 1), jnp.float32),
          jax.ShapeDtypeStruct((1, E), jnp.int32),
      ),
  )(x, gate_w)


@functools.cache
def _sc_kernels():
  mesh = plsc.VectorSubcoreMesh(core_axis_name="c", subcore_axis_name="s")
  scratch = [
      pltpu.VMEM((_ROWS_PER_W,), jnp.int32),
      pltpu.VMEM((_ROWS_PER_W, D), jnp.float32),
      pltpu.SemaphoreType.DMA,
  ]

  @functools.partial(
      pl.kernel,
      out_type=jax.ShapeDtypeStruct((M_PAD, D), jnp.float32),
      mesh=mesh,
      scratch_types=scratch,
  )
  def dispatch(x_hbm, idx_hbm, xs_hbm, idx_v, rows_v, sem):
    wid = lax.axis_index("s") * _SC_CORES + lax.axis_index("c")
    base = wid * _ROWS_PER_W
    pltpu.sync_copy(idx_hbm.at[pl.ds(base, _ROWS_PER_W)], idx_v)
    pltpu.sync_copy(x_hbm.at[pl.ds(base, _ROWS_PER_W)], rows_v)
    pltpu.async_copy(rows_v, xs_hbm.at[idx_v], sem).wait()

  @functools.partial(
      pl.kernel,
      out_type=jax.ShapeDtypeStruct((S, D), jnp.float32),
      mesh=mesh,
      scratch_types=scratch,
  )
  def combine_gather(routed_hbm, idx_hbm, out_hbm, idx_v, rows_v, sem):
    wid = lax.axis_index("s") * _SC_CORES + lax.axis_index("c")
    base = wid * _ROWS_PER_W
    pltpu.sync_copy(idx_hbm.at[pl.ds(base, _ROWS_PER_W)], idx_v)
    pltpu.async_copy(routed_hbm.at[idx_v], rows_v, sem).wait()
    pltpu.sync_copy(rows_v, out_hbm.at[pl.ds(base, _ROWS_PER_W)])

  return dispatch, combine_gather


def _gelu(h):
  return 0.5 * h * (1.0 + lax.erf(h * (1.0 / math.sqrt(2.0))))


def _ffn_block(x, w1, b1, w2, b2):
  h = lax.dot_general(x, w1, (((1,), (1,)), ((), ())),
                      preferred_element_type=jnp.float32) + b1
  h = _gelu(h)
  return lax.dot_general(h, w2, (((1,), (1,)), ((), ())),
                         preferred_element_type=jnp.float32) + b2


def _routed_body(be_ref, xs_ref, w1_ref, b1_ref, w2_ref, b2_ref, o_ref):
  o_ref[...] = _ffn_block(xs_ref[...], w1_ref[0], b1_ref[0], w2_ref[0],
                          b2_ref[0])


def _routed_ffn(be, xs, w1, b1, w2, b2):
  grid_spec = pltpu.PrefetchScalarGridSpec(
      num_scalar_prefetch=1,
      grid=(NB,),
      in_specs=[
          pl.BlockSpec((T, D), lambda mb, be: (mb, 0)),
          pl.BlockSpec((1, DFF, D), lambda mb, be: (be[mb], 0, 0)),
          pl.BlockSpec((1, 1, DFF), lambda mb, be: (be[mb], 0, 0)),
          pl.BlockSpec((1, D, DFF), lambda mb, be: (be[mb], 0, 0)),
          pl.BlockSpec((1, 1, D), lambda mb, be: (be[mb], 0, 0)),
      ],
      out_specs=pl.BlockSpec((T, D), lambda mb, be: (mb, 0)),
  )
  return pl.pallas_call(
      _routed_body,
      grid_spec=grid_spec,
      out_shape=jax.ShapeDtypeStruct((M_PAD, D), jnp.float32),
  )(be, xs, w1, b1, w2, b2)


def _shared_body(x_ref, w1_ref, b1_ref, w2_ref, b2_ref, o_ref):
  o_ref[...] = _ffn_block(x_ref[...], w1_ref[...], b1_ref[...], w2_ref[...],
                          b2_ref[...])


def _shared_ffn(x, sw1, sb1, sw2, sb2):
  return pl.pallas_call(
      _shared_body,
      grid=(S // T,),
      in_specs=[
          pl.BlockSpec((T, D), lambda mb: (mb, 0)),
          pl.BlockSpec((DFF, D), lambda mb: (0, 0)),
          pl.BlockSpec((1, DFF), lambda mb: (0, 0)),
          pl.BlockSpec((D, DFF), lambda mb: (0, 0)),
          pl.BlockSpec((1, D), lambda mb: (0, 0)),
      ],
      out_specs=pl.BlockSpec((T, D), lambda mb: (mb, 0)),
      out_shape=jax.ShapeDtypeStruct((S, D), jnp.float32),
  )(x, sw1, sb1, sw2, sb2)


def _final_body(wt_ref, g_ref, s_ref, o_ref):
  o_ref[...] = wt_ref[...] * g_ref[...] + s_ref[...]


def _final(wt, gathered, shared):
  return pl.pallas_call(
      _final_body,
      out_shape=jax.ShapeDtypeStruct((S, D), jnp.float32),
  )(wt, gathered, shared)


def kernel(hidden_states, gate_w, w1, b1, w2, b2, sw1, sb1, sw2, sb2):
  x = hidden_states.reshape(S, D)
  dst2, wt2, cnt2 = _gate(x, gate_w)
  dst = dst2.reshape(S)
  counts = cnt2.reshape(E)

  pc = ((counts + (T - 1)) // T) * T
  starts = jnp.cumsum(pc) - pc
  mb_off = jnp.arange(NB, dtype=jnp.int32) * T
  be = jnp.sum((mb_off[:, None] >= starts[None, :]).astype(jnp.int32),
               axis=1) - 1
  be = jnp.clip(be, 0, E - 1).astype(jnp.int32)

  dispatch, combine_gather = _sc_kernels()
  xs = dispatch(x, dst)
  routed = _routed_ffn(be, xs, w1, b1.reshape(E, 1, DFF),
                       w2, b2.reshape(E, 1, D))
  shared = _shared_ffn(x, sw1, sb1.reshape(1, DFF), sw2, sb2.reshape(1, D))
  gathered = combine_gather(routed, dst)
  out = _final(wt2, gathered, shared)
  return out.reshape(hidden_states.shape)
